# baseline (speedup 1.0000x reference)
import functools

import jax
import jax.numpy as jnp
from jax import lax
from jax.experimental import pallas as pl
from jax.experimental.pallas import tpu as pltpu
from jax.experimental.pallas import tpu_sc as plsc

NTOKEN = 100000
NTOKEN_C = 1000
EMB_DIM = 300
C_EMB_DIM = 64

B0, B1 = 4096, 50
B_TOTAL = B0 * B1
NC, NS = 2, 16
NW = NC * NS
B_PER_W = B_TOTAL // NW
CHUNK = 128
N_CHUNKS = B_PER_W // CHUNK
EMB_PAD = 304


def _make_embed_kernel():
    mesh = plsc.VectorSubcoreMesh(core_axis_name="c", subcore_axis_name="s")

    @functools.partial(
        pl.kernel,
        mesh=mesh,
        out_type=(
            jax.ShapeDtypeStruct((B_TOTAL, EMB_PAD), jnp.float32),
            jax.ShapeDtypeStruct((B_TOTAL, C_EMB_DIM), jnp.float32),
        ),
        scratch_types=[
            pltpu.VMEM((N_CHUNKS, CHUNK), jnp.int32),
            pltpu.VMEM((N_CHUNKS, CHUNK), jnp.int32),
            pltpu.VMEM((CHUNK, EMB_PAD), jnp.float32),
            pltpu.VMEM((CHUNK, C_EMB_DIM), jnp.float32),
            pltpu.SemaphoreType.DMA,
        ],
        compiler_params=pltpu.CompilerParams(use_tc_tiling_on_sc=False),
    )
    def embed_kernel(x_hbm, xc_hbm, emb_hbm, cemb_hbm, out_hbm, outc_hbm,
                     idx_v, idxc_v, rows_v, crows_v, sem):
        wid = lax.axis_index("s") * NC + lax.axis_index("c")
        base = wid * B_PER_W
        pltpu.sync_copy(x_hbm.at[wid], idx_v)
        pltpu.sync_copy(xc_hbm.at[wid], idxc_v)

        def body(j, carry):
            row0 = base + j * CHUNK
            pltpu.async_copy(emb_hbm.at[idx_v.at[j]], rows_v, sem).wait()
            pltpu.sync_copy(rows_v, out_hbm.at[pl.ds(row0, CHUNK)])
            pltpu.async_copy(cemb_hbm.at[idxc_v.at[j]], crows_v, sem).wait()
            pltpu.sync_copy(crows_v, outc_hbm.at[pl.ds(row0, CHUNK)])
            return carry

        lax.fori_loop(0, N_CHUNKS, body, 0)

    return embed_kernel


_embed = _make_embed_kernel()


def kernel(x, x_c, emb_W, c_emb_W):
    xf = x.reshape(NW, N_CHUNKS, CHUNK).astype(jnp.int32)
    xcf = x_c.reshape(NW, N_CHUNKS, CHUNK).astype(jnp.int32)
    emb_p = jnp.pad(emb_W, ((0, 0), (0, EMB_PAD - EMB_DIM)))
    out, outc = _embed(xf, xcf, emb_p, c_emb_W)
    return (out[:, :EMB_DIM].reshape(B0, B1, EMB_DIM),
            outc.reshape(B0, B1, C_EMB_DIM))


# speedup vs baseline: 1.2546x; 1.2546x over previous
import functools

import jax
import jax.numpy as jnp
from jax import lax
from jax.experimental import pallas as pl
from jax.experimental.pallas import tpu as pltpu
from jax.experimental.pallas import tpu_sc as plsc

NTOKEN = 100000
NTOKEN_C = 1000
EMB_DIM = 300
C_EMB_DIM = 64
EMB_PAD = 384
C_EMB_PAD = 128

B0, B1 = 4096, 50
B_TOTAL = B0 * B1
NC, NS = 2, 16
NW = NC * NS
B_PER_W = B_TOTAL // NW
CHUNK = 128
N_CHUNKS = B_PER_W // CHUNK


def _make_embed_kernel():
    mesh = plsc.VectorSubcoreMesh(core_axis_name="c", subcore_axis_name="s")

    @functools.partial(
        pl.kernel,
        mesh=mesh,
        out_type=(
            jax.ShapeDtypeStruct((B_TOTAL, EMB_PAD), jnp.float32),
            jax.ShapeDtypeStruct((B_TOTAL, C_EMB_PAD), jnp.float32),
        ),
        scratch_types=[
            pltpu.VMEM((N_CHUNKS, CHUNK), jnp.int32),
            pltpu.VMEM((N_CHUNKS, CHUNK), jnp.int32),
            pltpu.VMEM((CHUNK, EMB_PAD), jnp.float32),
            pltpu.VMEM((CHUNK, C_EMB_PAD), jnp.float32),
            pltpu.SemaphoreType.DMA,
        ],
    )
    def embed_kernel(x_hbm, xc_hbm, emb_hbm, cemb_hbm, out_hbm, outc_hbm,
                     idx_v, idxc_v, rows_v, crows_v, sem):
        wid = lax.axis_index("s") * NC + lax.axis_index("c")
        base = wid * B_PER_W
        pltpu.sync_copy(x_hbm.at[wid], idx_v)
        pltpu.sync_copy(xc_hbm.at[wid], idxc_v)

        def body(j, carry):
            row0 = base + j * CHUNK
            pltpu.async_copy(emb_hbm.at[idx_v.at[j]], rows_v, sem).wait()
            pltpu.sync_copy(rows_v, out_hbm.at[pl.ds(row0, CHUNK)])
            pltpu.async_copy(cemb_hbm.at[idxc_v.at[j]], crows_v, sem).wait()
            pltpu.sync_copy(crows_v, outc_hbm.at[pl.ds(row0, CHUNK)])
            return carry

        lax.fori_loop(0, N_CHUNKS, body, 0)

    return embed_kernel


_embed = _make_embed_kernel()


def kernel(x, x_c, emb_W, c_emb_W):
    xf = x.reshape(NW, N_CHUNKS, CHUNK).astype(jnp.int32)
    xcf = x_c.reshape(NW, N_CHUNKS, CHUNK).astype(jnp.int32)
    emb_p = jnp.pad(emb_W, ((0, 0), (0, EMB_PAD - EMB_DIM)))
    cemb_p = jnp.pad(c_emb_W, ((0, 0), (0, C_EMB_PAD - C_EMB_DIM)))
    out, outc = _embed(xf, xcf, emb_p, cemb_p)
    return (out[:, :EMB_DIM].reshape(B0, B1, EMB_DIM),
            outc[:, :C_EMB_DIM].reshape(B0, B1, C_EMB_DIM))


# speedup vs baseline: 1.2833x; 1.0229x over previous
import functools

import jax
import jax.numpy as jnp
from jax import lax
from jax.experimental import pallas as pl
from jax.experimental.pallas import tpu as pltpu
from jax.experimental.pallas import tpu_sc as plsc

NTOKEN = 100000
NTOKEN_C = 1000
EMB_DIM = 300
C_EMB_DIM = 64
EMB_PAD = 384
C_EMB_PAD = 128

B0, B1 = 4096, 50
B_TOTAL = B0 * B1
NC, NS = 2, 16
NW = NC * NS
B_PER_W = B_TOTAL // NW
CHUNK = 128
N_CHUNKS = B_PER_W // CHUNK


def _make_embed_kernel():
    mesh = plsc.VectorSubcoreMesh(core_axis_name="c", subcore_axis_name="s")

    @functools.partial(
        pl.kernel,
        mesh=mesh,
        out_type=(
            jax.ShapeDtypeStruct((B_TOTAL, EMB_PAD), jnp.float32),
            jax.ShapeDtypeStruct((B_TOTAL, C_EMB_PAD), jnp.float32),
        ),
        scratch_types=[
            pltpu.VMEM((N_CHUNKS, CHUNK), jnp.int32),
            pltpu.VMEM((N_CHUNKS, CHUNK), jnp.int32),
            pltpu.VMEM((CHUNK, EMB_PAD), jnp.float32),
            pltpu.VMEM((CHUNK, C_EMB_PAD), jnp.float32),
            pltpu.SemaphoreType.DMA,
        ],
    )
    def embed_kernel(x_hbm, xc_hbm, emb_hbm, cemb_hbm, out_hbm, outc_hbm,
                     idx_v, idxc_v, rows_v, crows_v, sem):
        wid = lax.axis_index("s") * NC + lax.axis_index("c")
        base = wid * B_PER_W
        pltpu.sync_copy(x_hbm.at[wid], idx_v)
        pltpu.sync_copy(xc_hbm.at[wid], idxc_v)

        def body(j, carry):
            row0 = base + j * CHUNK
            pltpu.async_copy(emb_hbm.at[idx_v.at[j]], rows_v, sem).wait()
            pltpu.sync_copy(rows_v, out_hbm.at[pl.ds(row0, CHUNK)])
            pltpu.async_copy(cemb_hbm.at[idxc_v.at[j]], crows_v, sem).wait()
            pltpu.sync_copy(crows_v, outc_hbm.at[pl.ds(row0, CHUNK)])
            return carry

        lax.fori_loop(0, N_CHUNKS, body, 0)

    return embed_kernel


_embed = _make_embed_kernel()


_PAD_ROWS = 1000


def _pad_body(i_ref, o_ref):
    o_ref[:, :EMB_DIM] = i_ref[...]
    o_ref[:, EMB_DIM:] = jnp.zeros((_PAD_ROWS, EMB_PAD - EMB_DIM), jnp.float32)


_pad_emb = pl.pallas_call(
    _pad_body,
    grid=(NTOKEN // _PAD_ROWS,),
    in_specs=[pl.BlockSpec((_PAD_ROWS, EMB_DIM), lambda i: (i, 0))],
    out_specs=pl.BlockSpec((_PAD_ROWS, EMB_PAD), lambda i: (i, 0)),
    out_shape=jax.ShapeDtypeStruct((NTOKEN, EMB_PAD), jnp.float32),
)

_SLC_SEQ = 8


def _make_slice(din_pad, dout):
    def body(i_ref, o_ref):
        for s in range(_SLC_SEQ):
            o_ref[0, s] = i_ref[0, pl.ds(s * B1, B1), :dout]

    return pl.pallas_call(
        body,
        grid=(B0 // _SLC_SEQ,),
        in_specs=[pl.BlockSpec((1, _SLC_SEQ * B1, din_pad), lambda i: (i, 0, 0))],
        out_specs=pl.BlockSpec((1, _SLC_SEQ, B1, dout), lambda i: (i, 0, 0, 0)),
        out_shape=jax.ShapeDtypeStruct((B0 // _SLC_SEQ, _SLC_SEQ, B1, dout),
                                       jnp.float32),
    )


_slice_emb = _make_slice(EMB_PAD, EMB_DIM)
_slice_cemb = _make_slice(C_EMB_PAD, C_EMB_DIM)


def kernel(x, x_c, emb_W, c_emb_W):
    xf = x.reshape(NW, N_CHUNKS, CHUNK).astype(jnp.int32)
    xcf = x_c.reshape(NW, N_CHUNKS, CHUNK).astype(jnp.int32)
    emb_p = _pad_emb(emb_W)
    cemb_p = jnp.pad(c_emb_W, ((0, 0), (0, C_EMB_PAD - C_EMB_DIM)))
    out, outc = _embed(xf, xcf, emb_p, cemb_p)
    out = _slice_emb(out.reshape(B0 // _SLC_SEQ, _SLC_SEQ * B1, EMB_PAD))
    outc = _slice_cemb(outc.reshape(B0 // _SLC_SEQ, _SLC_SEQ * B1, C_EMB_PAD))
    return (out.reshape(B0, B1, EMB_DIM), outc.reshape(B0, B1, C_EMB_DIM))
